# baseline (speedup 1.0000x reference)
import functools

import jax
import jax.numpy as jnp
from jax import lax
from jax.experimental import pallas as pl
from jax.experimental.pallas import tpu as pltpu
from jax.experimental.pallas import tpu_sc as plsc

N = 10000
E = 320000
D = 128
HD = 64
DG = 16
NP = 10016
NC = 2
NS = 16
EPT = E // NS
CH = 256
CPW = EPT // CH
TAIL = EPT - CPW * CH
RPT = NP // NS
XRPT = N // NS
BLK = 2504


def _sc_aggregate(x, edge_index):
    mesh = plsc.VectorSubcoreMesh(core_axis_name="c", subcore_axis_name="s")

    @functools.partial(
        pl.kernel,
        out_type=(jax.ShapeDtypeStruct((NC, NP, HD), jnp.float32),
                  jax.ShapeDtypeStruct((NP, DG), jnp.float32)),
        mesh=mesh,
        compiler_params=pltpu.CompilerParams(use_tc_tiling_on_sc=False),
        scratch_types=[
            pltpu.VMEM((2, 2, CH), jnp.int32),
            pltpu.VMEM((2, TAIL), jnp.int32),
            pltpu.VMEM((2, CH, HD), jnp.float32),
            pltpu.VMEM((CH, DG), jnp.float32),
            pltpu.VMEM_SHARED((N, HD), jnp.float32),
            pltpu.VMEM_SHARED((NP, HD), jnp.float32),
            pltpu.VMEM_SHARED((NP, DG), jnp.float32),
            pltpu.SemaphoreType.DMA,
            pltpu.SemaphoreType.DMA,
            pltpu.SemaphoreType.DMA,
            pltpu.SemaphoreType.DMA,
        ],
    )
    def agg(x_hbm, ed_hbm, out_hbm, deg_hbm,
            ibr, ibt, bufr, ones_v, xs, acc, dacc,
            sem0, sem1, semi0, semi1):
        c = lax.axis_index("c")
        s = lax.axis_index("s")
        base = s * RPT
        eoff = s * EPT

        ibs = (ibr.at[0], ibr.at[1])
        bufs = (bufr.at[0], bufr.at[1])
        sems = (sem0, sem1)
        semis = (semi0, semi1)

        zv = jnp.zeros((16,), jnp.float32)

        @pl.loop(0, CH)
        def _(r):
            for j in range(HD // 16):
                bufr[0, r, pl.ds(16 * j, 16)] = zv

        for j in range(RPT // CH):
            pltpu.sync_copy(bufs[0], acc.at[pl.ds(base + j * CH, CH)])
        rem = RPT % CH
        if rem:
            pltpu.sync_copy(bufs[0].at[pl.ds(0, rem)],
                            acc.at[pl.ds(base + (RPT // CH) * CH, rem)])

        @pl.loop(0, CH)
        def _(r):
            ones_v[r, :] = zv
        for j in range(RPT // CH):
            pltpu.sync_copy(ones_v, dacc.at[pl.ds(base + j * CH, CH)])
        if rem:
            pltpu.sync_copy(ones_v.at[pl.ds(0, rem)],
                            dacc.at[pl.ds(base + (RPT // CH) * CH, rem)])

        xr0 = s * XRPT
        for (ro, rn) in ((0, CH), (CH, CH), (2 * CH, XRPT - 2 * CH)):
            pltpu.sync_copy(
                x_hbm.at[pl.ds(xr0 + ro, rn), pl.ds(c * HD, HD)],
                bufs[1].at[pl.ds(0, rn)])
            pltpu.sync_copy(bufs[1].at[pl.ds(0, rn)],
                            xs.at[pl.ds(xr0 + ro, rn)])

        ov = jnp.ones((16,), jnp.float32)

        @pl.loop(0, CH)
        def _(r):
            ones_v[r, :] = ov

        plsc.subcore_barrier()

        def start_idx(chunk, p):
            off = eoff + chunk * CH
            pltpu.async_copy(ed_hbm.at[:, pl.ds(off, CH)], ibs[p], semis[p])

        def wait_idx(p):
            pltpu.make_async_copy(ed_hbm.at[:, pl.ds(0, CH)], ibs[p],
                                  semis[p]).wait()

        def start_gather(p):
            pltpu.async_copy(xs.at[ibs[p].at[0]], bufs[p], sems[p])

        def wait_gather(p):
            pltpu.make_async_copy(xs.at[ibs[p].at[0]], bufs[p],
                                  sems[p]).wait()

        pltpu.sync_copy(ed_hbm.at[:, pl.ds(eoff, CH)], ibs[0])
        start_gather(0)
        start_idx(1, 1)

        @pl.loop(0, CPW // 2)
        def _(k):
            for half in range(2):
                x_, y = half, 1 - half
                t = 2 * k + half
                wait_gather(x_)

                @pl.when(t < CPW - 1)
                def _():
                    wait_idx(y)
                    start_gather(y)

                pltpu.sync_copy(bufs[x_], acc.at[ibs[x_].at[1]], add=True)

                @pl.when(c == 0)
                def _():
                    pltpu.sync_copy(ones_v, dacc.at[ibs[x_].at[1]], add=True)

                @pl.when(t < CPW - 2)
                def _():
                    start_idx(t + 2, x_)

        if TAIL:
            toff = eoff + CPW * CH
            pltpu.sync_copy(ed_hbm.at[:, pl.ds(toff, TAIL)], ibt)
            pltpu.sync_copy(xs.at[ibt.at[0]], bufs[0].at[pl.ds(0, TAIL)])
            pltpu.sync_copy(bufs[0].at[pl.ds(0, TAIL)],
                            acc.at[ibt.at[1]], add=True)

            @pl.when(c == 0)
            def _():
                pltpu.sync_copy(ones_v.at[pl.ds(0, TAIL)],
                                dacc.at[ibt.at[1]], add=True)

        plsc.subcore_barrier()

        for j in range(RPT // CH):
            r = base + j * CH
            pltpu.sync_copy(acc.at[pl.ds(r, CH)], bufs[0])
            pltpu.sync_copy(bufs[0], out_hbm.at[c, pl.ds(r, CH)])
        if rem:
            r = base + (RPT // CH) * CH
            pltpu.sync_copy(acc.at[pl.ds(r, rem)], bufs[1].at[pl.ds(0, rem)])
            pltpu.sync_copy(bufs[1].at[pl.ds(0, rem)],
                            out_hbm.at[c, pl.ds(r, rem)])

        @pl.when(c == 0)
        def _():
            for j in range(RPT // CH):
                r = base + j * CH
                pltpu.sync_copy(dacc.at[pl.ds(r, CH)], ones_v)
                pltpu.sync_copy(ones_v, deg_hbm.at[pl.ds(r, CH)])
            if rem:
                r = base + (RPT // CH) * CH
                pltpu.sync_copy(dacc.at[pl.ds(r, rem)],
                                ones_v.at[pl.ds(0, rem)])
                pltpu.sync_copy(ones_v.at[pl.ds(0, rem)],
                                deg_hbm.at[pl.ds(r, rem)])

    return agg(x, edge_index)


def _tc_finish(parts, degs, wt, b2):
    def body(p_ref, d_ref, w_ref, b_ref, o_ref):
        deg = jnp.maximum(d_ref[:, :1], 1.0)
        h0 = p_ref[0] / deg
        h1 = p_ref[1] / deg
        dn = (((1,), (0,)), ((), ()))
        o_ref[...] = (
            lax.dot_general(h0, w_ref[0], dn,
                            preferred_element_type=jnp.float32)
            + lax.dot_general(h1, w_ref[1], dn,
                              preferred_element_type=jnp.float32)
            + b_ref[...])

    return pl.pallas_call(
        body,
        grid=(NP // BLK,),
        in_specs=[
            pl.BlockSpec((NC, BLK, HD), lambda i: (0, i, 0)),
            pl.BlockSpec((BLK, DG), lambda i: (i, 0)),
            pl.BlockSpec((NC, HD, D), lambda i: (0, 0, 0)),
            pl.BlockSpec((1, D), lambda i: (0, 0)),
        ],
        out_specs=pl.BlockSpec((BLK, D), lambda i: (i, 0)),
        out_shape=jax.ShapeDtypeStruct((N, D), jnp.float32),
    )(parts, degs, wt, b2)


def kernel(x, edge_index, W, b):
    x = x.astype(jnp.float32)
    ed = edge_index.astype(jnp.int32)
    parts, degs = _sc_aggregate(x, ed)
    wt = W.T.reshape(NC, HD, D)
    return _tc_finish(parts, degs, wt, b.reshape(1, D))


# speedup vs baseline: 1.3309x; 1.3309x over previous
import functools

import jax
import jax.numpy as jnp
from jax import lax
from jax.experimental import pallas as pl
from jax.experimental.pallas import tpu as pltpu
from jax.experimental.pallas import tpu_sc as plsc

N = 10000
E = 320000
D = 128
DG = 16
NP = 10048
NC = 2
NS = 16
NW = NC * NS
EPW = E // NW
CH = 128
CPW = EPW // CH
TAIL = EPW - CPW * CH
RPT = NP // NS
BLK = 1256


def _sc_aggregate(x, edge_index):
    mesh = plsc.VectorSubcoreMesh(core_axis_name="c", subcore_axis_name="s")

    @functools.partial(
        pl.kernel,
        out_type=(jax.ShapeDtypeStruct((NC, NP, D), jnp.float32),
                  jax.ShapeDtypeStruct((NC, NP, DG), jnp.float32)),
        mesh=mesh,
        compiler_params=pltpu.CompilerParams(use_tc_tiling_on_sc=False),
        scratch_types=[
            pltpu.VMEM((2, CH), jnp.int32),
            pltpu.VMEM((2, CH), jnp.int32),
            pltpu.VMEM((2, TAIL), jnp.int32),
            pltpu.VMEM((CH, D), jnp.float32),
            pltpu.VMEM((CH, D), jnp.float32),
            pltpu.VMEM((CH, DG), jnp.float32),
            pltpu.VMEM((CH, DG), jnp.float32),
            pltpu.VMEM_SHARED((NP, D), jnp.float32),
            pltpu.VMEM_SHARED((NP, DG), jnp.float32),
            pltpu.SemaphoreType.DMA,
            pltpu.SemaphoreType.DMA,
            pltpu.SemaphoreType.DMA,
            pltpu.SemaphoreType.DMA,
            pltpu.SemaphoreType.DMA,
            pltpu.SemaphoreType.DMA,
        ],
    )
    def agg(x_hbm, ed_hbm, out_hbm, deg_hbm,
            ib0, ib1, ibt, buf0, buf1, ones_v, dstage, acc, dacc,
            sem0, sem1, semi0, semi1, sems_f, sems_d):
        c = lax.axis_index("c")
        s = lax.axis_index("s")
        w = c * NS + s
        base = s * RPT
        eoff = w * EPW

        ibs = (ib0, ib1)
        bufs = (buf0, buf1)
        sems = (sem0, sem1)
        semis = (semi0, semi1)

        zv = jnp.zeros((16,), jnp.float32)

        @pl.loop(0, CH)
        def _(r):
            for j in range(D // 16):
                buf0[r, pl.ds(16 * j, 16)] = zv

        for j in range(RPT // CH):
            pltpu.sync_copy(buf0, acc.at[pl.ds(base + j * CH, CH)])
        rem = RPT % CH
        if rem:
            pltpu.sync_copy(buf0.at[pl.ds(0, rem)],
                            acc.at[pl.ds(base + (RPT // CH) * CH, rem)])

        @pl.loop(0, CH)
        def _(r):
            ones_v[r, :] = zv
        for j in range(RPT // CH):
            pltpu.sync_copy(ones_v, dacc.at[pl.ds(base + j * CH, CH)])
        if rem:
            pltpu.sync_copy(ones_v.at[pl.ds(0, rem)],
                            dacc.at[pl.ds(base + (RPT // CH) * CH, rem)])

        ov = jnp.ones((16,), jnp.float32)

        @pl.loop(0, CH)
        def _(r):
            ones_v[r, :] = ov

        plsc.subcore_barrier()

        def start_idx(chunk, p):
            off = eoff + chunk * CH
            pltpu.async_copy(ed_hbm.at[:, pl.ds(off, CH)], ibs[p], semis[p])

        def wait_idx(p):
            pltpu.make_async_copy(ed_hbm.at[:, pl.ds(0, CH)], ibs[p],
                                  semis[p]).wait()

        def start_gather(p):
            pltpu.async_copy(x_hbm.at[ibs[p].at[0]], bufs[p], sems[p])

        def wait_gather(p):
            pltpu.make_async_copy(x_hbm.at[ibs[p].at[0]], bufs[p],
                                  sems[p]).wait()

        def wait_scatters(p):
            pltpu.make_async_copy(bufs[p], acc.at[ibs[p].at[1]],
                                  sems_f).wait()
            pltpu.make_async_copy(ones_v, dacc.at[ibs[p].at[1]],
                                  sems_d).wait()

        pltpu.sync_copy(ed_hbm.at[:, pl.ds(eoff, CH)], ib0)
        start_gather(0)
        start_idx(1, 1)

        @pl.loop(0, CPW // 2)
        def _(k):
            for half in range(2):
                x_, y = half, 1 - half
                t = 2 * k + half
                wait_gather(x_)

                @pl.when(t < CPW - 1)
                def _():
                    wait_idx(y)
                    start_gather(y)

                pltpu.async_copy(bufs[x_], acc.at[ibs[x_].at[1]], sems_f,
                                 add=True)
                pltpu.async_copy(ones_v, dacc.at[ibs[x_].at[1]], sems_d,
                                 add=True)
                wait_scatters(x_)

                @pl.when(t < CPW - 2)
                def _():
                    start_idx(t + 2, x_)

        if TAIL:
            toff = eoff + CPW * CH
            pltpu.sync_copy(ed_hbm.at[:, pl.ds(toff, TAIL)], ibt)
            pltpu.sync_copy(x_hbm.at[ibt.at[0]], buf0.at[pl.ds(0, TAIL)])
            pltpu.sync_copy(buf0.at[pl.ds(0, TAIL)],
                            acc.at[ibt.at[1]], add=True)
            pltpu.sync_copy(ones_v.at[pl.ds(0, TAIL)],
                            dacc.at[ibt.at[1]], add=True)

        plsc.subcore_barrier()

        for j in range(RPT // CH):
            r = base + j * CH
            pltpu.sync_copy(acc.at[pl.ds(r, CH)], buf0)
            pltpu.sync_copy(buf0, out_hbm.at[c, pl.ds(r, CH)])
            pltpu.sync_copy(dacc.at[pl.ds(r, CH)], dstage)
            pltpu.sync_copy(dstage, deg_hbm.at[c, pl.ds(r, CH)])
        if rem:
            r = base + (RPT // CH) * CH
            pltpu.sync_copy(acc.at[pl.ds(r, rem)], buf1.at[pl.ds(0, rem)])
            pltpu.sync_copy(buf1.at[pl.ds(0, rem)], out_hbm.at[c, pl.ds(r, rem)])
            pltpu.sync_copy(dacc.at[pl.ds(r, rem)], dstage.at[pl.ds(0, rem)])
            pltpu.sync_copy(dstage.at[pl.ds(0, rem)],
                            deg_hbm.at[c, pl.ds(r, rem)])

    return agg(x, edge_index)


def _tc_finish(parts, degs, W, b2):
    def body(p_ref, d_ref, w_ref, b_ref, o_ref):
        p = p_ref[0] + p_ref[1]
        dsum = d_ref[0] + d_ref[1]
        deg = jnp.maximum(dsum[:, :1], 1.0)
        h = p / deg
        o_ref[...] = lax.dot_general(
            h, w_ref[...], (((1,), (1,)), ((), ())),
            preferred_element_type=jnp.float32) + b_ref[...]

    return pl.pallas_call(
        body,
        grid=(NP // BLK,),
        in_specs=[
            pl.BlockSpec((NC, BLK, D), lambda i: (0, i, 0)),
            pl.BlockSpec((NC, BLK, DG), lambda i: (0, i, 0)),
            pl.BlockSpec((D, D), lambda i: (0, 0)),
            pl.BlockSpec((1, D), lambda i: (0, 0)),
        ],
        out_specs=pl.BlockSpec((BLK, D), lambda i: (i, 0)),
        out_shape=jax.ShapeDtypeStruct((N, D), jnp.float32),
    )(parts, degs, W, b2)


def kernel(x, edge_index, W, b):
    x = x.astype(jnp.float32)
    ed = edge_index.astype(jnp.int32)
    parts, degs = _sc_aggregate(x, ed)
    return _tc_finish(parts, degs, W, b.reshape(1, D))
